# baseline (speedup 1.0000x reference)
import functools

import jax
import jax.numpy as jnp
from jax import lax
from jax.experimental import pallas as pl
from jax.experimental.pallas import tpu as pltpu
from jax.experimental.pallas import tpu_sc as plsc

NC = 2
NS = 16
NW = NC * NS
LANES = 16
CHUNK = 128


@functools.cache
def _build_sc_call(B, C, H, W, NPAD):
    NPIX = H * W
    SPAN = (B * NPAD) // NW
    NCHUNK = SPAN // CHUNK
    WPB = NW // B

    mesh = plsc.VectorSubcoreMesh(core_axis_name="c", subcore_axis_name="s")

    @functools.partial(
        pl.kernel,
        out_type=jax.ShapeDtypeStruct((B * NPAD, C), jnp.float32),
        mesh=mesh,
        scratch_types=[
            pltpu.VMEM((CHUNK,), jnp.float32),
            pltpu.VMEM((CHUNK,), jnp.float32),
            pltpu.VMEM((CHUNK,), jnp.int32),
            pltpu.VMEM((CHUNK,), jnp.int32),
            pltpu.VMEM((CHUNK,), jnp.int32),
            pltpu.VMEM((CHUNK,), jnp.int32),
            pltpu.VMEM((CHUNK,), jnp.float32),
            pltpu.VMEM((CHUNK,), jnp.float32),
            pltpu.VMEM((CHUNK,), jnp.float32),
            pltpu.VMEM((CHUNK,), jnp.float32),
            pltpu.VMEM((CHUNK, C), jnp.float32),
            pltpu.VMEM((CHUNK, C), jnp.float32),
            pltpu.VMEM((CHUNK, C), jnp.float32),
            pltpu.VMEM((CHUNK, C), jnp.float32),
            pltpu.VMEM((CHUNK, C), jnp.float32),
            pltpu.SemaphoreType.DMA,
        ],
        compiler_params=pltpu.CompilerParams(needs_layout_passes=False,
                                             use_tc_tiling_on_sc=False),
    )
    def sc_sample(table, xs, ys, out, xv, yv, i00, i01, i10, i11,
                  w00, w01, w10, w11, v00, v01, v10, v11, outb, sem):
        cid = lax.axis_index("c")
        sid = lax.axis_index("s")
        wid = sid * NC + cid
        base_row = wid * SPAN
        tab_off = (wid // WPB) * NPIX

        iota = lax.broadcasted_iota(jnp.int32, (LANES,), 0)

        def chunk_body(ci, carry):
            rbase = base_row + ci * CHUNK
            pltpu.sync_copy(xs.at[pl.ds(rbase, CHUNK)], xv)
            pltpu.sync_copy(ys.at[pl.ds(rbase, CHUNK)], yv)
            for g in range(CHUNK // LANES):
                sl = pl.ds(g * LANES, LANES)
                x = xv[sl]
                y = yv[sl]
                fx = jnp.minimum(jnp.maximum((x + 1.0) * 0.5 * (W - 1.0), 0.0),
                                 W - 1.0)
                fy = jnp.minimum(jnp.maximum((y + 1.0) * 0.5 * (H - 1.0), 0.0),
                                 H - 1.0)
                ix0 = fx.astype(jnp.int32)
                iy0 = fy.astype(jnp.int32)
                wx1 = fx - ix0.astype(jnp.float32)
                wy1 = fy - iy0.astype(jnp.float32)
                wx0 = 1.0 - wx1
                wy0 = 1.0 - wy1
                dx = jnp.where(ix0 < W - 1, 1, 0)
                dy = jnp.where(iy0 < H - 1, W, 0)
                p00 = tab_off + iy0 * W + ix0
                i00[sl] = p00
                i01[sl] = p00 + dx
                i10[sl] = p00 + dy
                i11[sl] = p00 + dy + dx
                w00[sl] = wy0 * wx0
                w01[sl] = wy0 * wx1
                w10[sl] = wy1 * wx0
                w11[sl] = wy1 * wx1
            c0 = pltpu.async_copy(table.at[i00], v00, sem)
            c1 = pltpu.async_copy(table.at[i01], v01, sem)
            c2 = pltpu.async_copy(table.at[i10], v10, sem)
            c3 = pltpu.async_copy(table.at[i11], v11, sem)
            c0.wait()
            c1.wait()
            c2.wait()
            c3.wait()
            zeros16 = jnp.zeros((LANES,), jnp.int32)

            def row_body(r, carry2):
                bidx = zeros16 + r
                b00 = plsc.load_gather(w00, [bidx])
                b01 = plsc.load_gather(w01, [bidx])
                b10 = plsc.load_gather(w10, [bidx])
                b11 = plsc.load_gather(w11, [bidx])
                for j in range(C // LANES):
                    sl = pl.ds(j * LANES, LANES)
                    t0 = v00[r, sl]
                    t1 = v01[r, sl]
                    t2 = v10[r, sl]
                    t3 = v11[r, sl]
                    outb[r, sl] = t0 * b00 + t1 * b01 + t2 * b10 + t3 * b11
                return carry2

            lax.fori_loop(0, CHUNK, row_body, 0)
            pltpu.sync_copy(outb, out.at[pl.ds(rbase, CHUNK)])
            return carry

        lax.fori_loop(0, NCHUNK, chunk_body, 0)

    return sc_sample


def kernel(image_features, vertices):
    B, C, H, W = image_features.shape
    N = vertices.shape[1]
    step = (NW // B) * CHUNK
    NPAD = ((N + step - 1) // step) * step

    table = jnp.transpose(image_features, (0, 2, 3, 1)).reshape(B * H * W, C)
    v = jnp.pad(vertices, ((0, 0), (0, NPAD - N), (0, 0)))
    xs = v[..., 0].reshape(-1)
    ys = v[..., 1].reshape(-1)

    sc_sample = _build_sc_call(B, C, H, W, NPAD)
    out = sc_sample(table, xs, ys)
    return out.reshape(B, NPAD, C)[:, :N, :]
